# baseline (speedup 1.0000x reference)
import math

import jax
import jax.numpy as jnp
from jax.experimental import pallas as pl
from jax.experimental.pallas import tpu as pltpu

_SQRT_HALF = 0.7071067811865476


def _round_up(x, m):
    return (x + m - 1) // m * m


def _gelu_exact(x):
    return 0.5 * x * (1.0 + jax.lax.erf(x * jnp.float32(_SQRT_HALF)))


def _make_body(lps, kk, out_ch, in_ch, hid, o_pad):
    inv_scale = float(1.0 / (math.sqrt(hid) * out_ch))
    D = in_ch * kk

    def _body(x_ref, w_ref, b_ref, o_ref, wT_ref):
        for j in range(lps):
            b = b_ref[j]

            wT_ref[...] = jnp.swapaxes(w_ref[j], 0, 1).reshape(
                D, hid // 128, 128)

            h = None
            for t in range(kk):
                wt = wT_ref[pl.Slice(t, in_ch, kk), :, :].reshape(
                    in_ch, hid)
                p = jax.lax.dot_general(
                    x_ref[j, t], wt,
                    dimension_numbers=(((1,), (0,)), ((), ())),
                    preferred_element_type=jnp.float32)
                h = p if h is None else h + p
            h = _gelu_exact(h + b)

            if o_pad > out_ch:
                row = jax.lax.broadcasted_iota(jnp.int32, h.shape, 0)
                h = jnp.where(row < out_ch, h, 0.0)

            s = jax.lax.dot_general(
                h, h, dimension_numbers=(((1,), (1,)), ((), ())),
                preferred_element_type=jnp.float32)

            r = jnp.sum(jnp.abs(s), axis=0, keepdims=True) * inv_scale

            if o_pad > out_ch:
                lane = jax.lax.broadcasted_iota(jnp.int32, (1, o_pad), 1)
                valid = lane < out_ch
                r_min = jnp.min(jnp.where(valid, r, jnp.inf), keepdims=True)
                r_max = jnp.max(jnp.where(valid, r, -jnp.inf), keepdims=True)
                rn = jnp.where(valid, (r - r_min) /
                               (r_max - r_min + jnp.float32(1e-8)), 0.0)
            else:
                r_min = jnp.min(r, keepdims=True)
                r_max = jnp.max(r, keepdims=True)
                rn = (r - r_min) / (r_max - r_min + jnp.float32(1e-8))

            o_ref[j] = rn

    return _body


def kernel(conv_weights, fc_weights, fc_biases):
    L, out_ch, in_ch, kh, kw = conv_weights.shape
    kk = kh * kw
    hid = fc_weights.shape[1]
    D = in_ch * kk

    o_pad = max(128, _round_up(out_ch, 128))
    lps = 1
    grid = L // lps

    x = jnp.transpose(conv_weights, (0, 3, 4, 1, 2)).astype(jnp.float32)
    x = x.reshape(L, kk, out_ch, in_ch)
    w = fc_weights.astype(jnp.float32)
    b = fc_biases.reshape(L, 1, hid).astype(jnp.float32)

    if o_pad > out_ch:
        x = jnp.pad(x, ((0, 0), (0, 0), (0, o_pad - out_ch), (0, 0)))

    out = pl.pallas_call(
        _make_body(lps, kk, out_ch, in_ch, hid, o_pad),
        out_shape=jax.ShapeDtypeStruct((L, 1, o_pad), jnp.float32),
        grid_spec=pltpu.PrefetchScalarGridSpec(
            num_scalar_prefetch=0,
            grid=(grid,),
            in_specs=[
                pl.BlockSpec((lps, kk, o_pad, in_ch), lambda l: (l, 0, 0, 0)),
                pl.BlockSpec((lps, hid, D), lambda l: (l, 0, 0)),
                pl.BlockSpec((lps, 1, hid), lambda l: (l, 0, 0)),
            ],
            out_specs=pl.BlockSpec((lps, 1, o_pad), lambda l: (l, 0, 0)),
            scratch_shapes=[pltpu.VMEM((D, hid // 128, 128), jnp.float32)],
        ),
        compiler_params=pltpu.CompilerParams(
            dimension_semantics=("parallel",),
            vmem_limit_bytes=100 * 1024 * 1024),
    )(x, w, b)

    return out[:, 0, :out_ch]


# speedup vs baseline: 1.1275x; 1.1275x over previous
import math

import jax
import jax.numpy as jnp
from jax.experimental import pallas as pl
from jax.experimental.pallas import tpu as pltpu

_SQRT_HALF = 0.7071067811865476


def _round_up(x, m):
    return (x + m - 1) // m * m


def _gelu_exact(x):
    return 0.5 * x * (1.0 + jax.lax.erf(x * jnp.float32(_SQRT_HALF)))


def _make_body(lps, kk, out_ch, in_ch, hid, o_pad):
    inv_scale = float(1.0 / (math.sqrt(hid) * out_ch))
    D = in_ch * kk

    def _body(x_ref, w_ref, b_ref, o_ref, wT_ref):
        for j in range(lps):
            b = b_ref[j]

            wT_ref[...] = jnp.swapaxes(w_ref[j], 0, 1).reshape(
                D, hid // 128, 128)

            h = None
            for t in range(kk):
                wt = wT_ref[pl.Slice(t, in_ch, kk), :, :].reshape(
                    in_ch, hid)
                p = jax.lax.dot_general(
                    x_ref[j, t], wt,
                    dimension_numbers=(((1,), (0,)), ((), ())),
                    preferred_element_type=jnp.float32)
                h = p if h is None else h + p
            h = _gelu_exact(h + b)

            if o_pad > out_ch:
                row = jax.lax.broadcasted_iota(jnp.int32, h.shape, 0)
                h = jnp.where(row < out_ch, h, 0.0)

            s = jax.lax.dot_general(
                h, h, dimension_numbers=(((1,), (1,)), ((), ())),
                preferred_element_type=jnp.float32)

            r = jnp.sum(jnp.abs(s), axis=0, keepdims=True) * inv_scale

            if o_pad > out_ch:
                lane = jax.lax.broadcasted_iota(jnp.int32, (1, o_pad), 1)
                valid = lane < out_ch
                r_min = jnp.min(jnp.where(valid, r, jnp.inf), keepdims=True)
                r_max = jnp.max(jnp.where(valid, r, -jnp.inf), keepdims=True)
                rn = jnp.where(valid, (r - r_min) /
                               (r_max - r_min + jnp.float32(1e-8)), 0.0)
            else:
                r_min = jnp.min(r, keepdims=True)
                r_max = jnp.max(r, keepdims=True)
                rn = (r - r_min) / (r_max - r_min + jnp.float32(1e-8))

            o_ref[j] = rn

    return _body


def kernel(conv_weights, fc_weights, fc_biases):
    L, out_ch, in_ch, kh, kw = conv_weights.shape
    kk = kh * kw
    hid = fc_weights.shape[1]
    D = in_ch * kk

    o_pad = max(128, _round_up(out_ch, 128))
    lps = 4 if L % 4 == 0 else (2 if L % 2 == 0 else 1)
    grid = L // lps

    x = jnp.transpose(conv_weights, (0, 3, 4, 1, 2)).astype(jnp.float32)
    x = x.reshape(L, kk, out_ch, in_ch)
    w = fc_weights.astype(jnp.float32)
    b = fc_biases.reshape(L, 1, hid).astype(jnp.float32)

    if o_pad > out_ch:
        x = jnp.pad(x, ((0, 0), (0, 0), (0, o_pad - out_ch), (0, 0)))

    out = pl.pallas_call(
        _make_body(lps, kk, out_ch, in_ch, hid, o_pad),
        out_shape=jax.ShapeDtypeStruct((L, 1, o_pad), jnp.float32),
        grid_spec=pltpu.PrefetchScalarGridSpec(
            num_scalar_prefetch=0,
            grid=(grid,),
            in_specs=[
                pl.BlockSpec((lps, kk, o_pad, in_ch), lambda l: (l, 0, 0, 0)),
                pl.BlockSpec((lps, hid, D), lambda l: (l, 0, 0)),
                pl.BlockSpec((lps, 1, hid), lambda l: (l, 0, 0)),
            ],
            out_specs=pl.BlockSpec((lps, 1, o_pad), lambda l: (l, 0, 0)),
            scratch_shapes=[pltpu.VMEM((D, hid // 128, 128), jnp.float32)],
        ),
        compiler_params=pltpu.CompilerParams(
            dimension_semantics=("parallel",),
            vmem_limit_bytes=100 * 1024 * 1024),
    )(x, w, b)

    return out[:, 0, :out_ch]
